# baseline (speedup 1.0000x reference)
import jax
import jax.numpy as jnp
from jax.experimental import pallas as pl

MARGIN_ = 10.0
ALPHA_ = 1.0
EPS_ = 1e-05

BR = 512
NS = 8


def _body(*refs):
    (*x_refs, t_ref, out_ref) = refs
    i = pl.program_id(0)
    c = x_refs[0].shape[1]

    cols = jax.lax.broadcasted_iota(jnp.int32, (BR, c), 1)
    zero = jnp.float32(0.0)
    s_lse, s_pick, s_relu, s_x = zero, zero, zero, zero
    for k, x_ref in enumerate(x_refs):
        x = x_ref[...]
        t = t_ref[0, k, :]
        m = jnp.max(x, axis=1, keepdims=True)
        se = jnp.sum(jnp.exp(x - m), axis=1)
        s_lse += jnp.sum(m[:, 0] + jnp.log(se))
        s_relu += jnp.sum(jnp.maximum((m - MARGIN_) - x, 0.0))
        s_x += jnp.sum(x)
        s_pick += jnp.sum(jnp.where(cols == t[:, None], x, 0.0))

    lane = jax.lax.broadcasted_iota(jnp.int32, (1, 128), 1)
    part = (jnp.where(lane == 0, s_lse, 0.0)
            + jnp.where(lane == 1, s_pick, 0.0)
            + jnp.where(lane == 2, s_relu, 0.0)
            + jnp.where(lane == 3, s_x, 0.0))

    @pl.when(i == 0)
    def _():
        out_ref[...] = jnp.zeros_like(out_ref)

    out_ref[...] += part


def kernel(inputs, targets):
    n, c = inputs.shape
    grid = n // (NS * BR)
    t3 = targets.astype(jnp.int32).reshape(grid, NS, BR)
    in_specs = [pl.BlockSpec((BR, c), (lambda i, k=k: (NS * i + k, 0)))
                for k in range(NS)]
    in_specs.append(pl.BlockSpec((1, NS, BR), lambda i: (i, 0, 0)))
    out = pl.pallas_call(
        _body,
        grid=(grid,),
        in_specs=in_specs,
        out_specs=pl.BlockSpec((1, 128), lambda i: (0, 0)),
        out_shape=jax.ShapeDtypeStruct((1, 128), jnp.float32),
    )(*([inputs] * NS), t3)

    s_lse, s_pick = out[0, 0], out[0, 1]
    s_relu, s_x = out[0, 2], out[0, 3]
    loss_ce = (s_lse - s_pick) / n
    loss_margin = s_relu / (n * c)
    dice = (2.0 * s_pick + EPS_) / ((n + s_x) + EPS_)
    loss_dice = 1.0 - dice
    loss = loss_ce + loss_dice + ALPHA_ * loss_margin
    return (loss, loss_ce, loss_margin, loss_dice)


# speedup vs baseline: 1.1976x; 1.1976x over previous
import jax
import jax.numpy as jnp
from jax.experimental import pallas as pl

MARGIN_ = 10.0
ALPHA_ = 1.0
EPS_ = 1e-05

BR = 1024
NS = 4


def _body(a_ref, b_ref, c_ref, d_ref, t_ref, out_ref):
    i = pl.program_id(0)
    c = a_ref.shape[1]

    cols = jax.lax.broadcasted_iota(jnp.int32, (BR, c), 1)
    zero = jnp.float32(0.0)
    s_lse, s_pick, s_relu, s_x = zero, zero, zero, zero
    for k, x_ref in enumerate((a_ref, b_ref, c_ref, d_ref)):
        x = x_ref[...]
        t = t_ref[0, k, :]
        m = jnp.max(x, axis=1, keepdims=True)
        se = jnp.sum(jnp.exp(x - m), axis=1)
        s_lse += jnp.sum(m[:, 0] + jnp.log(se))
        s_relu += jnp.sum(jnp.maximum((m - MARGIN_) - x, 0.0))
        s_x += jnp.sum(x)
        s_pick += jnp.sum(jnp.where(cols == t[:, None], x, 0.0))

    lane = jax.lax.broadcasted_iota(jnp.int32, (1, 128), 1)
    part = (jnp.where(lane == 0, s_lse, 0.0)
            + jnp.where(lane == 1, s_pick, 0.0)
            + jnp.where(lane == 2, s_relu, 0.0)
            + jnp.where(lane == 3, s_x, 0.0))

    @pl.when(i == 0)
    def _():
        out_ref[...] = jnp.zeros_like(out_ref)

    out_ref[...] += part


def kernel(inputs, targets):
    n, c = inputs.shape
    grid = n // (NS * BR)
    t3 = targets.astype(jnp.int32).reshape(grid, NS, BR)
    in_specs = [pl.BlockSpec((BR, c), (lambda i, k=k: (NS * i + k, 0)))
                for k in range(NS)]
    in_specs.append(pl.BlockSpec((1, NS, BR), lambda i: (i, 0, 0)))
    out = pl.pallas_call(
        _body,
        grid=(grid,),
        in_specs=in_specs,
        out_specs=pl.BlockSpec((1, 128), lambda i: (0, 0)),
        out_shape=jax.ShapeDtypeStruct((1, 128), jnp.float32),
    )(*([inputs] * NS), t3)

    s_lse, s_pick = out[0, 0], out[0, 1]
    s_relu, s_x = out[0, 2], out[0, 3]
    loss_ce = (s_lse - s_pick) / n
    loss_margin = s_relu / (n * c)
    dice = (2.0 * s_pick + EPS_) / ((n + s_x) + EPS_)
    loss_dice = 1.0 - dice
    loss = loss_ce + loss_dice + ALPHA_ * loss_margin
    return (loss, loss_ce, loss_margin, loss_dice)
